# baseline (speedup 1.0000x reference)
import functools

import jax
import jax.numpy as jnp
from jax import lax
from jax.experimental import pallas as pl
from jax.experimental.pallas import tpu as pltpu
from jax.experimental.pallas import tpu_sc as plsc

_NC = 2
_NS = 16
_NW = _NC * _NS
_LN = 16

_DPAD = 16
_CHUNK = 40
_GBLK = 50


def _mm_body(x_ref, w_ref, o_ref):
    o_ref[...] = jnp.dot(x_ref[...], w_ref[...],
                         preferred_element_type=jnp.float32)


def _edge_mm(ee, w, block):
    e, d = ee.shape
    return pl.pallas_call(
        _mm_body,
        grid=(e // block,),
        in_specs=[pl.BlockSpec((block, d), lambda i: (i, 0)),
                  pl.BlockSpec((d, d), lambda i: (0, 0))],
        out_specs=pl.BlockSpec((block, d), lambda i: (i, 0)),
        out_shape=jax.ShapeDtypeStruct((e, d), jnp.float32),
    )(ee, w)


def _edge_mm_part(ee, w, block, wchunks, j0, nj):
    e, d = ee.shape

    def imap(wi, j):
        return (wi * wchunks + j0 + j, 0)

    return pl.pallas_call(
        _mm_body,
        grid=(e // (block * wchunks), nj),
        in_specs=[pl.BlockSpec((block, d), imap),
                  pl.BlockSpec((d, d), lambda wi, j: (0, 0))],
        out_specs=pl.BlockSpec((block, d), imap),
        out_shape=jax.ShapeDtypeStruct((e, d), jnp.float32),
    )(ee, w)


def _pre_body(x_ref, wa_ref, wb_ref, b_ref, p_ref, q_ref):
    x = x_ref[...]
    p_ref[...] = jnp.dot(x, wa_ref[...], preferred_element_type=jnp.float32)
    q_ref[...] = (jnp.dot(x, wb_ref[...], preferred_element_type=jnp.float32)
                  + b_ref[...])


def _pre(x, wa, wb, b, block):
    n, d = x.shape
    shp = jax.ShapeDtypeStruct((n, d), jnp.float32)
    return pl.pallas_call(
        _pre_body,
        grid=(n // block,),
        in_specs=[pl.BlockSpec((block, d), lambda i: (i, 0)),
                  pl.BlockSpec((d, d), lambda i: (0, 0)),
                  pl.BlockSpec((d, d), lambda i: (0, 0)),
                  pl.BlockSpec((1, d), lambda i: (0, 0))],
        out_specs=[pl.BlockSpec((block, d), lambda i: (i, 0)),
                   pl.BlockSpec((block, d), lambda i: (i, 0))],
        out_shape=[shp, shp],
    )(x, wa, wb, b)


def _post_mid_body(sa_ref, sb_ref, w2_ref, wa_ref, wb_ref, b_ref,
                   p_ref, q_ref):
    s = sa_ref[0] + sa_ref[1] + sb_ref[0] + sb_ref[1]
    x = jnp.maximum(
        jnp.dot(s, w2_ref[...], preferred_element_type=jnp.float32), 0.0)
    p_ref[...] = jnp.dot(x, wa_ref[...], preferred_element_type=jnp.float32)
    q_ref[...] = (jnp.dot(x, wb_ref[...], preferred_element_type=jnp.float32)
                  + b_ref[...])


def _post_mid(sa, sb, w2h, wa, wb, b, n, block):
    da = sa.shape[2]
    d = w2h.shape[1]
    shp = jax.ShapeDtypeStruct((n, d), jnp.float32)
    sspec = pl.BlockSpec((_NC, block, da), lambda i: (0, i, 0))
    return pl.pallas_call(
        _post_mid_body,
        grid=(n // block,),
        in_specs=[sspec, sspec,
                  pl.BlockSpec((da, d), lambda i: (0, 0)),
                  pl.BlockSpec((d, d), lambda i: (0, 0)),
                  pl.BlockSpec((d, d), lambda i: (0, 0)),
                  pl.BlockSpec((1, d), lambda i: (0, 0))],
        out_specs=[pl.BlockSpec((block, d), lambda i: (i, 0)),
                   pl.BlockSpec((block, d), lambda i: (i, 0))],
        out_shape=[shp, shp],
    )(sa, sb, w2h, wa, wb, b)


def _post_end_body(s_ref, w2_ref, o_ref):
    o_ref[...] = jnp.maximum(
        jnp.dot(s_ref[0] + s_ref[1], w2_ref[...],
                preferred_element_type=jnp.float32), 0.0)


def _post_end(shat, w2h, n, block):
    da = shat.shape[2]
    d = w2h.shape[1]
    return pl.pallas_call(
        _post_end_body,
        grid=(n // block,),
        in_specs=[pl.BlockSpec((_NC, block, da), lambda i: (0, i, 0)),
                  pl.BlockSpec((da, d), lambda i: (0, 0))],
        out_specs=pl.BlockSpec((block, d), lambda i: (i, 0)),
        out_shape=jax.ShapeDtypeStruct((n, d), jnp.float32),
    )(shat, w2h)


@functools.cache
def _make_edge_kernel(n, d, e, c0, ncht):
    da = d + _DPAD
    ew = e // _NW
    c = _CHUNK
    ni = ew // c
    g = _GBLK
    nb = ncht // g
    npr = g // 2
    rpt = n // _NS // 8 * 8
    rlast = n - (_NS - 1) * rpt

    mesh = plsc.VectorSubcoreMesh(core_axis_name="c", subcore_axis_name="s")

    @functools.partial(
        pl.kernel,
        out_type=jax.ShapeDtypeStruct((_NC, n, da), jnp.float32),
        mesh=mesh,
        compiler_params=pltpu.CompilerParams(use_tc_tiling_on_sc=False),
        scratch_types=[
            pltpu.VMEM((g, c), jnp.int32),
            pltpu.VMEM((g, c), jnp.int32),
            pltpu.VMEM((2, c, d), jnp.float32),
            pltpu.VMEM((2, c, d), jnp.float32),
            pltpu.VMEM((2, c, d), jnp.float32),
            pltpu.VMEM((c, da), jnp.float32),
            pltpu.VMEM_SHARED((n, da), jnp.float32),
            pltpu.SemaphoreType.DMA,
            pltpu.SemaphoreType.DMA,
            pltpu.SemaphoreType.DMA,
            pltpu.SemaphoreType.DMA,
            pltpu.SemaphoreType.DMA,
            pltpu.SemaphoreType.DMA,
        ],
    )
    def edge_kernel(p_hbm, q_hbm, ec_hbm, src_hbm, dst_hbm, out_hbm,
                    src_v, dst_v, pr, qr, er, hh, shat,
                    sp0, sq0, se0, sp1, sq1, se1):
        cid = lax.axis_index("c")
        sid = lax.axis_index("s")
        wid = sid * _NC + cid
        sems = ((sp0, sq0, se0), (sp1, sq1, se1))

        zero = jnp.zeros((_LN,), jnp.float32)
        one0 = jnp.where(lax.iota(jnp.int32, _LN) == 0, 1.0, 0.0)

        def zrow(r, _):
            for k in range(da // _LN):
                hh[r, pl.ds(k * _LN, _LN)] = zero
            return 0
        lax.fori_loop(0, c, zrow, 0)

        row0 = sid * rpt

        @pl.when(sid < _NS - 1)
        def _():
            for z in range(rpt // c):
                pltpu.sync_copy(hh, shat.at[pl.ds(row0 + z * c, c)])
            if rpt % c:
                pltpu.sync_copy(hh.at[pl.ds(0, rpt % c)],
                                shat.at[pl.ds(row0 + rpt // c * c, rpt % c)])

        @pl.when(sid == _NS - 1)
        def _():
            for z in range(rlast // c):
                pltpu.sync_copy(hh, shat.at[pl.ds(row0 + z * c, c)])
            if rlast % c:
                pltpu.sync_copy(hh.at[pl.ds(0, rlast % c)],
                                shat.at[pl.ds(row0 + rlast // c * c,
                                              rlast % c)])

        def hrow(r, _):
            hh[r, pl.ds(d, _LN)] = one0
            return 0
        lax.fori_loop(0, c, hrow, 0)
        plsc.subcore_barrier()

        def fire(buf, k):
            sp, sq, _ = sems[buf]
            pltpu.async_copy(p_hbm.at[src_v.at[k]], pr.at[buf], sp)
            pltpu.async_copy(q_hbm.at[dst_v.at[k]], qr.at[buf], sq)

        def fire_ec(buf, gb, k):
            _, _, se = sems[buf]
            pltpu.async_copy(
                ec_hbm.at[pl.ds((wid * ni + c0 + gb * g + k) * c, c)],
                er.at[buf], se)

        def wait(buf):
            sp, sq, se = sems[buf]
            pltpu.make_async_copy(p_hbm.at[src_v.at[0]], pr.at[buf], sp).wait()
            pltpu.make_async_copy(q_hbm.at[dst_v.at[0]], qr.at[buf], sq).wait()
            pltpu.make_async_copy(ec_hbm.at[pl.ds(0, c)], er.at[buf],
                                  se).wait()

        def compute(buf):
            @plsc.parallel_loop(0, c, step=1, unroll=4)
            def _(r):
                for kk in range(d // _LN):
                    sl = pl.ds(kk * _LN, _LN)
                    hh[r, sl] = jnp.maximum(
                        pr[buf, r, sl] + qr[buf, r, sl] + er[buf, r, sl], 0.0)

        def block(gb, _):
            pltpu.sync_copy(src_hbm.at[wid, pl.ds(c0 + gb * g, g)], src_v)
            pltpu.sync_copy(dst_hbm.at[wid, pl.ds(c0 + gb * g, g)], dst_v)
            fire(0, 0)
            fire_ec(0, gb, 0)
            fire(1, 1)
            fire_ec(1, gb, 1)

            def pair(jp, _):
                ka = 2 * jp
                wait(0)
                compute(0)
                pltpu.sync_copy(hh, shat.at[dst_v.at[ka]], add=True)

                @pl.when(jp < npr - 1)
                def _():
                    fire(0, ka + 2)
                    fire_ec(0, gb, ka + 2)

                wait(1)
                compute(1)
                pltpu.sync_copy(hh, shat.at[dst_v.at[ka + 1]], add=True)

                @pl.when(jp < npr - 1)
                def _():
                    fire(1, ka + 3)
                    fire_ec(1, gb, ka + 3)
                return 0
            lax.fori_loop(0, npr, pair, 0)
            return 0
        lax.fori_loop(0, nb, block, 0)

        plsc.subcore_barrier()

        @pl.when(sid < _NS - 1)
        def _():
            pltpu.sync_copy(shat.at[pl.ds(row0, rpt)],
                            out_hbm.at[cid, pl.ds(row0, rpt)])

        @pl.when(sid == _NS - 1)
        def _():
            pltpu.sync_copy(shat.at[pl.ds(row0, rlast)],
                            out_hbm.at[cid, pl.ds(row0, rlast)])

    return edge_kernel


def kernel(node_embeds, edge_index, edge_embeds, W1, b1, W2, b2):
    n, d = node_embeds.shape
    e = edge_index.shape[1]
    nl = W1.shape[0]
    ew = e // _NW
    ni = ew // _CHUNK

    src_r = edge_index[0].reshape(_NW, ni, _CHUNK)
    dst_r = edge_index[1].reshape(_NW, ni, _CHUNK)

    csp = ni * 2 // 5 // _GBLK * _GBLK
    edge_a = _make_edge_kernel(n, d, e, 0, csp)
    edge_b = _make_edge_kernel(n, d, e, csp, ni - csp)
    edge_full = _make_edge_kernel(n, d, e, 0, ni)

    wch = ni * _CHUNK // 2000
    jsp = csp * _CHUNK // 2000

    zpad = jnp.zeros((_DPAD - 1, d), jnp.float32)
    w2h = [jnp.concatenate([W2[l], b2[l][None], zpad], axis=0)
           for l in range(nl)]

    p, q = _pre(node_embeds, W1[0, :d], W1[0, d:2 * d], b1[0][None], 2000)
    ec0a = _edge_mm_part(edge_embeds, W1[0, 2 * d:], 2000, wch, 0, jsp)
    ec0b = _edge_mm_part(edge_embeds, W1[0, 2 * d:], 2000, wch, jsp,
                         wch - jsp)
    ec1 = _edge_mm(edge_embeds, W1[1, 2 * d:], 2560)

    sa = edge_a(p, q, ec0a, src_r, dst_r)
    sb = edge_b(p, q, ec0b, src_r, dst_r)
    p, q = _post_mid(sa, sb, w2h[0], W1[1, :d], W1[1, d:2 * d],
                     b1[1][None], n, 2000)
    s1 = edge_full(p, q, ec1, src_r, dst_r)
    return _post_end(s1, w2h[1], n, 2000)


# speedup vs baseline: 1.0318x; 1.0318x over previous
import functools

import jax
import jax.numpy as jnp
from jax import lax
from jax.experimental import pallas as pl
from jax.experimental.pallas import tpu as pltpu
from jax.experimental.pallas import tpu_sc as plsc

_NC = 2
_NS = 16
_NW = _NC * _NS
_LN = 16

_DPAD = 16
_CHUNK = 40
_GBLK = 50


def _mm_body(x_ref, w_ref, o_ref):
    o_ref[...] = jnp.dot(x_ref[...], w_ref[...],
                         preferred_element_type=jnp.float32)


def _edge_mm(ee, w, block):
    e, d = ee.shape
    return pl.pallas_call(
        _mm_body,
        grid=(e // block,),
        in_specs=[pl.BlockSpec((block, d), lambda i: (i, 0)),
                  pl.BlockSpec((d, d), lambda i: (0, 0))],
        out_specs=pl.BlockSpec((block, d), lambda i: (i, 0)),
        out_shape=jax.ShapeDtypeStruct((e, d), jnp.float32),
    )(ee, w)


def _edge_mm_part(ee, w, block, wchunks, j0, nj):
    e, d = ee.shape

    def imap(wi, j):
        return (wi * wchunks + j0 + j, 0)

    return pl.pallas_call(
        _mm_body,
        grid=(e // (block * wchunks), nj),
        in_specs=[pl.BlockSpec((block, d), imap),
                  pl.BlockSpec((d, d), lambda wi, j: (0, 0))],
        out_specs=pl.BlockSpec((block, d), imap),
        out_shape=jax.ShapeDtypeStruct((e, d), jnp.float32),
    )(ee, w)


def _pre_body(x_ref, wa_ref, wb_ref, b_ref, p_ref, q_ref):
    x = x_ref[...]
    p_ref[...] = jnp.dot(x, wa_ref[...], preferred_element_type=jnp.float32)
    q_ref[...] = (jnp.dot(x, wb_ref[...], preferred_element_type=jnp.float32)
                  + b_ref[...])


def _pre(x, wa, wb, b, block):
    n, d = x.shape
    shp = jax.ShapeDtypeStruct((n, d), jnp.float32)
    return pl.pallas_call(
        _pre_body,
        grid=(n // block,),
        in_specs=[pl.BlockSpec((block, d), lambda i: (i, 0)),
                  pl.BlockSpec((d, d), lambda i: (0, 0)),
                  pl.BlockSpec((d, d), lambda i: (0, 0)),
                  pl.BlockSpec((1, d), lambda i: (0, 0))],
        out_specs=[pl.BlockSpec((block, d), lambda i: (i, 0)),
                   pl.BlockSpec((block, d), lambda i: (i, 0))],
        out_shape=[shp, shp],
    )(x, wa, wb, b)


def _post_mid_body(sa_ref, sb_ref, w2_ref, wa_ref, wb_ref, b_ref,
                   p_ref, q_ref):
    s = sa_ref[0] + sa_ref[1] + sb_ref[0] + sb_ref[1]
    x = jnp.maximum(
        jnp.dot(s, w2_ref[...], preferred_element_type=jnp.float32), 0.0)
    p_ref[...] = jnp.dot(x, wa_ref[...], preferred_element_type=jnp.float32)
    q_ref[...] = (jnp.dot(x, wb_ref[...], preferred_element_type=jnp.float32)
                  + b_ref[...])


def _post_mid(sa, sb, w2h, wa, wb, b, n, block):
    da = sa.shape[2]
    d = w2h.shape[1]
    shp = jax.ShapeDtypeStruct((n, d), jnp.float32)
    sspec = pl.BlockSpec((_NC, block, da), lambda i: (0, i, 0))
    return pl.pallas_call(
        _post_mid_body,
        grid=(n // block,),
        in_specs=[sspec, sspec,
                  pl.BlockSpec((da, d), lambda i: (0, 0)),
                  pl.BlockSpec((d, d), lambda i: (0, 0)),
                  pl.BlockSpec((d, d), lambda i: (0, 0)),
                  pl.BlockSpec((1, d), lambda i: (0, 0))],
        out_specs=[pl.BlockSpec((block, d), lambda i: (i, 0)),
                   pl.BlockSpec((block, d), lambda i: (i, 0))],
        out_shape=[shp, shp],
    )(sa, sb, w2h, wa, wb, b)


def _post_end_body(s_ref, w2_ref, o_ref):
    o_ref[...] = jnp.maximum(
        jnp.dot(s_ref[0] + s_ref[1], w2_ref[...],
                preferred_element_type=jnp.float32), 0.0)


def _post_end(shat, w2h, n, block):
    da = shat.shape[2]
    d = w2h.shape[1]
    return pl.pallas_call(
        _post_end_body,
        grid=(n // block,),
        in_specs=[pl.BlockSpec((_NC, block, da), lambda i: (0, i, 0)),
                  pl.BlockSpec((da, d), lambda i: (0, 0))],
        out_specs=pl.BlockSpec((block, d), lambda i: (i, 0)),
        out_shape=jax.ShapeDtypeStruct((n, d), jnp.float32),
    )(shat, w2h)


@functools.cache
def _make_edge_kernel(n, d, e, c0, ncht):
    da = d + _DPAD
    ew = e // _NW
    c = _CHUNK
    ni = ew // c
    g = _GBLK
    nb = ncht // g
    npr = g // 2
    rpt = n // _NS // 8 * 8
    rlast = n - (_NS - 1) * rpt

    mesh = plsc.VectorSubcoreMesh(core_axis_name="c", subcore_axis_name="s")

    @functools.partial(
        pl.kernel,
        out_type=jax.ShapeDtypeStruct((_NC, n, da), jnp.float32),
        mesh=mesh,
        compiler_params=pltpu.CompilerParams(use_tc_tiling_on_sc=False),
        scratch_types=[
            pltpu.VMEM((g, c), jnp.int32),
            pltpu.VMEM((g, c), jnp.int32),
            pltpu.VMEM((2, c, d), jnp.float32),
            pltpu.VMEM((2, c, d), jnp.float32),
            pltpu.VMEM((2, c, d), jnp.float32),
            pltpu.VMEM((c, da), jnp.float32),
            pltpu.VMEM_SHARED((n, da), jnp.float32),
            pltpu.SemaphoreType.DMA,
            pltpu.SemaphoreType.DMA,
            pltpu.SemaphoreType.DMA,
            pltpu.SemaphoreType.DMA,
            pltpu.SemaphoreType.DMA,
            pltpu.SemaphoreType.DMA,
        ],
    )
    def edge_kernel(p_hbm, q_hbm, ec_hbm, src_hbm, dst_hbm, out_hbm,
                    src_v, dst_v, pr, qr, er, hh, shat,
                    sp0, sq0, se0, sp1, sq1, se1):
        cid = lax.axis_index("c")
        sid = lax.axis_index("s")
        wid = sid * _NC + cid
        sems = ((sp0, sq0, se0), (sp1, sq1, se1))

        zero = jnp.zeros((_LN,), jnp.float32)
        one0 = jnp.where(lax.iota(jnp.int32, _LN) == 0, 1.0, 0.0)

        def zrow(r, _):
            for k in range(da // _LN):
                hh[r, pl.ds(k * _LN, _LN)] = zero
            return 0
        lax.fori_loop(0, c, zrow, 0)

        row0 = sid * rpt

        @pl.when(sid < _NS - 1)
        def _():
            for z in range(rpt // c):
                pltpu.sync_copy(hh, shat.at[pl.ds(row0 + z * c, c)])
            if rpt % c:
                pltpu.sync_copy(hh.at[pl.ds(0, rpt % c)],
                                shat.at[pl.ds(row0 + rpt // c * c, rpt % c)])

        @pl.when(sid == _NS - 1)
        def _():
            for z in range(rlast // c):
                pltpu.sync_copy(hh, shat.at[pl.ds(row0 + z * c, c)])
            if rlast % c:
                pltpu.sync_copy(hh.at[pl.ds(0, rlast % c)],
                                shat.at[pl.ds(row0 + rlast // c * c,
                                              rlast % c)])

        def hrow(r, _):
            hh[r, pl.ds(d, _LN)] = one0
            return 0
        lax.fori_loop(0, c, hrow, 0)
        plsc.subcore_barrier()

        def fire(buf, k):
            sp, sq, _ = sems[buf]
            pltpu.async_copy(p_hbm.at[src_v.at[k]], pr.at[buf], sp)
            pltpu.async_copy(q_hbm.at[dst_v.at[k]], qr.at[buf], sq)

        def fire_ec(buf, gb, k):
            _, _, se = sems[buf]
            pltpu.async_copy(
                ec_hbm.at[pl.ds((wid * ni + c0 + gb * g + k) * c, c)],
                er.at[buf], se)

        def wait(buf):
            sp, sq, se = sems[buf]
            pltpu.make_async_copy(p_hbm.at[src_v.at[0]], pr.at[buf], sp).wait()
            pltpu.make_async_copy(q_hbm.at[dst_v.at[0]], qr.at[buf], sq).wait()
            pltpu.make_async_copy(ec_hbm.at[pl.ds(0, c)], er.at[buf],
                                  se).wait()

        def compute(buf):
            @plsc.parallel_loop(0, c, step=1, unroll=4)
            def _(r):
                for kk in range(d // _LN):
                    sl = pl.ds(kk * _LN, _LN)
                    hh[r, sl] = jnp.maximum(
                        pr[buf, r, sl] + qr[buf, r, sl] + er[buf, r, sl], 0.0)

        def block(gb, _):
            pltpu.sync_copy(src_hbm.at[wid, pl.ds(c0 + gb * g, g)], src_v)
            pltpu.sync_copy(dst_hbm.at[wid, pl.ds(c0 + gb * g, g)], dst_v)
            fire(0, 0)
            fire_ec(0, gb, 0)
            fire(1, 1)
            fire_ec(1, gb, 1)

            def pair(jp, _):
                ka = 2 * jp
                wait(0)
                compute(0)
                pltpu.sync_copy(hh, shat.at[dst_v.at[ka]], add=True)

                @pl.when(jp < npr - 1)
                def _():
                    fire(0, ka + 2)
                    fire_ec(0, gb, ka + 2)

                wait(1)
                compute(1)
                pltpu.sync_copy(hh, shat.at[dst_v.at[ka + 1]], add=True)

                @pl.when(jp < npr - 1)
                def _():
                    fire(1, ka + 3)
                    fire_ec(1, gb, ka + 3)
                return 0
            lax.fori_loop(0, npr, pair, 0)
            return 0
        lax.fori_loop(0, nb, block, 0)

        plsc.subcore_barrier()

        @pl.when(sid < _NS - 1)
        def _():
            pltpu.sync_copy(shat.at[pl.ds(row0, rpt)],
                            out_hbm.at[cid, pl.ds(row0, rpt)])

        @pl.when(sid == _NS - 1)
        def _():
            pltpu.sync_copy(shat.at[pl.ds(row0, rlast)],
                            out_hbm.at[cid, pl.ds(row0, rlast)])

    return edge_kernel


def kernel(node_embeds, edge_index, edge_embeds, W1, b1, W2, b2):
    n, d = node_embeds.shape
    e = edge_index.shape[1]
    nl = W1.shape[0]
    ew = e // _NW
    ni = ew // _CHUNK

    src_r = edge_index[0].reshape(_NW, ni, _CHUNK)
    dst_r = edge_index[1].reshape(_NW, ni, _CHUNK)

    csp = ni * 3 // 5 // _GBLK * _GBLK
    edge_a = _make_edge_kernel(n, d, e, 0, csp)
    edge_b = _make_edge_kernel(n, d, e, csp, ni - csp)
    edge_full = _make_edge_kernel(n, d, e, 0, ni)

    wch = ni * _CHUNK // 2000
    jsp = csp * _CHUNK // 2000

    zpad = jnp.zeros((_DPAD - 1, d), jnp.float32)
    w2h = [jnp.concatenate([W2[l], b2[l][None], zpad], axis=0)
           for l in range(nl)]

    p, q = _pre(node_embeds, W1[0, :d], W1[0, d:2 * d], b1[0][None], 2000)
    ec0a = _edge_mm_part(edge_embeds, W1[0, 2 * d:], 2000, wch, 0, jsp)
    ec0b = _edge_mm_part(edge_embeds, W1[0, 2 * d:], 2000, wch, jsp,
                         wch - jsp)
    ec1 = _edge_mm(edge_embeds, W1[1, 2 * d:], 2560)

    sa = edge_a(p, q, ec0a, src_r, dst_r)
    sb = edge_b(p, q, ec0b, src_r, dst_r)
    p, q = _post_mid(sa, sb, w2h[0], W1[1, :d], W1[1, d:2 * d],
                     b1[1][None], n, 2000)
    s1 = edge_full(p, q, ec1, src_r, dst_r)
    return _post_end(s1, w2h[1], n, 2000)
